# baseline (speedup 1.0000x reference)
import functools

import jax
import jax.numpy as jnp
from jax import lax
from jax.experimental import pallas as pl
from jax.experimental.pallas import tpu as pltpu
from jax.experimental.pallas import tpu_sc as plsc

N = 8192
L = 16
NC = 2
NS = 16
NW = NC * NS

C_TC = 6656
SC_GEN = N - C_TC
NCH = SC_GEN // L
GT_PER_W = N // NW
NBLK_W = GT_PER_W // L
GSUB = 2
CGRP = 16

GT_TILE = 4096
GEN_TILE = 512

_F32_INF = 3.0e38

_GATHER_DNUMS = lax.GatherDimensionNumbers(
    offset_dims=(), collapsed_slice_dims=(0,), start_index_map=(0,))


def _permute16(x, idx):
    return lax.gather(x, idx[:, None], _GATHER_DNUMS, (1,),
                      indices_are_sorted=False, unique_indices=False,
                      mode=lax.GatherScatterMode.PROMISE_IN_BOUNDS)


def _all_lane_min(x, lane):
    for k in (8, 4, 2, 1):
        x = jnp.minimum(x, _permute16(x, lane ^ k))
    return x


def _rb_vec(x):
    u = lax.bitcast_convert_type(x, jnp.uint32)
    u = (u + jnp.uint32(0x7FFF) + ((u >> 16) & jnp.uint32(1))) \
        & jnp.uint32(0xFFFF0000)
    return lax.bitcast_convert_type(u, jnp.float32)


def _sc_body(gt_t, gen_t,
             gtp_out, genmin_out,
             rx_v, ry_v, rz_v,
             gs_s, a2_v, bx_v, by_v, bz_v, b2_v,
             gtp_v, gm_v):
    wid = lax.axis_index("s") * NC + lax.axis_index("c")
    base_t = wid * GT_PER_W

    pltpu.sync_copy(gen_t.at[pl.ds(0, 1), pl.ds(C_TC, SC_GEN)], rx_v)
    pltpu.sync_copy(gen_t.at[pl.ds(1, 1), pl.ds(C_TC, SC_GEN)], ry_v)
    pltpu.sync_copy(gen_t.at[pl.ds(2, 1), pl.ds(C_TC, SC_GEN)], rz_v)
    inf_vec = jnp.full((L,), _F32_INF, jnp.float32)

    def gen_stage(c, carry):
        sl = pl.ds(c * L, L)
        x, y, z = rx_v[0, sl], ry_v[0, sl], rz_v[0, sl]
        b2_v[sl] = x * x + y * y + z * z
        bx_v[sl] = _rb_vec(x)
        by_v[sl] = _rb_vec(y)
        bz_v[sl] = _rb_vec(z)
        gm_v[sl] = inf_vec
        return carry

    lax.fori_loop(0, NCH, gen_stage, 0)

    pltpu.sync_copy(gt_t.at[pl.ds(0, 1), pl.ds(base_t, GT_PER_W)],
                    rx_v.at[pl.ds(0, 1), pl.ds(0, GT_PER_W)])
    pltpu.sync_copy(gt_t.at[pl.ds(1, 1), pl.ds(base_t, GT_PER_W)],
                    ry_v.at[pl.ds(0, 1), pl.ds(0, GT_PER_W)])
    pltpu.sync_copy(gt_t.at[pl.ds(2, 1), pl.ds(base_t, GT_PER_W)],
                    rz_v.at[pl.ds(0, 1), pl.ds(0, GT_PER_W)])

    def gt_stage(c, carry):
        sl = pl.ds(c * L, L)
        x, y, z = rx_v[0, sl], ry_v[0, sl], rz_v[0, sl]
        a2 = x * x + y * y + z * z
        a2_v[sl] = a2
        gx = 2.0 * _rb_vec(x)
        gy = 2.0 * _rb_vec(y)
        gz = 2.0 * _rb_vec(z)
        base = c * L
        for g in range(L):
            gs_s[4 * (base + g) + 0] = gx[g]
            gs_s[4 * (base + g) + 1] = gy[g]
            gs_s[4 * (base + g) + 2] = gz[g]
            gs_s[4 * (base + g) + 3] = a2[g]
        return carry

    lax.fori_loop(0, NBLK_W, gt_stage, 0)

    lane = lax.iota(jnp.int32, L)

    def blk_body(blk, carry):
        base = blk * L
        a2v = a2_v[pl.ds(base, L)]
        r = inf_vec
        for s0 in range(0, L, GSUB):
            gs = range(s0, s0 + GSUB)
            ax = [gs_s[4 * (base + g) + 0] for g in gs]
            ay = [gs_s[4 * (base + g) + 1] for g in gs]
            az = [gs_s[4 * (base + g) + 2] for g in gs]
            a2 = [gs_s[4 * (base + g) + 3] for g in gs]

            def cg_body(cg, m):
                cbase = cg * (CGRP * L)
                for cc in range(CGRP):
                    off = cbase + cc * L
                    bx = bx_v[pl.ds(off, L)]
                    by = by_v[pl.ds(off, L)]
                    bz = bz_v[pl.ds(off, L)]
                    b2 = b2_v[pl.ds(off, L)]
                    gm = gm_v[pl.ds(off, L)]
                    for k in range(GSUB):
                        q = b2 - ax[k] * bx - ay[k] * by - az[k] * bz
                        gm = jnp.minimum(gm, q + a2[k])
                        m = tuple(
                            jnp.minimum(m[t], q) if t == k else m[t]
                            for t in range(GSUB))
                    gm_v[pl.ds(off, L)] = gm
                return m

            m = lax.fori_loop(0, NCH // CGRP, cg_body,
                              tuple(inf_vec for _ in range(GSUB)))
            for k in range(GSUB):
                r = jnp.where(lane == s0 + k, _all_lane_min(m[k], lane), r)
        gtp_v[pl.ds(base, L)] = r + a2v
        return carry

    lax.fori_loop(0, NBLK_W, blk_body, 0)

    pltpu.sync_copy(gtp_v, gtp_out.at[pl.ds(base_t, GT_PER_W)])
    pltpu.sync_copy(gm_v, genmin_out.at[wid])


def _sc_chamfer(gt_t, gen_t):
    mesh = plsc.VectorSubcoreMesh(core_axis_name="c", subcore_axis_name="s")
    f = pl.kernel(
        _sc_body,
        mesh=mesh,
        out_type=(
            jax.ShapeDtypeStruct((N,), jnp.float32),
            jax.ShapeDtypeStruct((NW, SC_GEN), jnp.float32),
        ),
        scratch_types=[
            pltpu.VMEM((1, SC_GEN), jnp.float32),
            pltpu.VMEM((1, SC_GEN), jnp.float32),
            pltpu.VMEM((1, SC_GEN), jnp.float32),
            pltpu.SMEM((4 * GT_PER_W,), jnp.float32),
            pltpu.VMEM((GT_PER_W,), jnp.float32),
            pltpu.VMEM((SC_GEN,), jnp.float32),
            pltpu.VMEM((SC_GEN,), jnp.float32),
            pltpu.VMEM((SC_GEN,), jnp.float32),
            pltpu.VMEM((SC_GEN,), jnp.float32),
            pltpu.VMEM((GT_PER_W,), jnp.float32),
            pltpu.VMEM((SC_GEN,), jnp.float32),
        ],
    )
    return f(gt_t, gen_t)


def _tc_body(gt2_ref, genT_ref, a2_ref, b2_ref, gtmin_ref, genmin_ref,
             acc_gt, acc_gen):
    j = pl.program_id(0)
    i = pl.program_id(1)
    ab2 = jnp.dot(gt2_ref[...], genT_ref[...],
                  preferred_element_type=jnp.float32)
    g8 = a2_ref[...]
    a2c = jnp.sum(g8 * g8, axis=1, keepdims=True)
    b2r = b2_ref[...]
    q = b2r - ab2
    p = a2c - ab2
    mq = jnp.min(q, axis=1, keepdims=True) + a2c
    mp = jnp.min(p, axis=0, keepdims=True) + b2r

    gt_sl = pl.ds(j * GT_TILE, GT_TILE)
    gen_sl = pl.ds(i * GEN_TILE, GEN_TILE)

    @pl.when(i == 0)
    def _():
        acc_gt[gt_sl, :] = mq

    @pl.when(i > 0)
    def _():
        acc_gt[gt_sl, :] = jnp.minimum(acc_gt[gt_sl, :], mq)

    @pl.when(j == 0)
    def _():
        acc_gen[:, gen_sl] = mp

    @pl.when(j > 0)
    def _():
        acc_gen[:, gen_sl] = jnp.minimum(acc_gen[:, gen_sl], mp)

    @pl.when((j == N // GT_TILE - 1) & (i == C_TC // GEN_TILE - 1))
    def _():
        gtmin_ref[...] = jnp.reshape(acc_gt[...], (64, 128))
        genmin_ref[...] = jnp.reshape(acc_gen[...], (C_TC // 128, 128))


def _tc_chamfer(gt2_pad, genT_pad, a2_col, b2_row):
    return pl.pallas_call(
        _tc_body,
        grid=(N // GT_TILE, C_TC // GEN_TILE),
        in_specs=[
            pl.BlockSpec((GT_TILE, 128), lambda j, i: (j, 0)),
            pl.BlockSpec((128, GEN_TILE), lambda j, i: (0, i)),
            pl.BlockSpec((GT_TILE, 8), lambda j, i: (j, 0)),
            pl.BlockSpec((1, GEN_TILE), lambda j, i: (0, i)),
        ],
        out_specs=[
            pl.BlockSpec((64, 128), lambda j, i: (0, 0)),
            pl.BlockSpec((C_TC // 128, 128), lambda j, i: (0, 0)),
        ],
        out_shape=[
            jax.ShapeDtypeStruct((64, 128), jnp.float32),
            jax.ShapeDtypeStruct((C_TC // 128, 128), jnp.float32),
        ],
        scratch_shapes=[
            pltpu.VMEM((N, 1), jnp.float32),
            pltpu.VMEM((1, C_TC), jnp.float32),
        ],
    )(gt2_pad, genT_pad, a2_col, b2_row)


def _combine_body(gtp_sc_ref, gtmin_tc_ref, gm_sc_ref, gm_tc_ref, out_ref):
    gt_full = jnp.minimum(gtp_sc_ref[...], gtmin_tc_ref[...])
    gen_sc = jnp.min(gm_sc_ref[...], axis=0)
    s = (jnp.sum(jnp.maximum(gt_full, 0.0))
         + jnp.sum(jnp.maximum(gen_sc, 0.0))
         + jnp.sum(jnp.maximum(gm_tc_ref[...], 0.0)))
    out_ref[0, 0] = s * (1.0 / N)


def _combine(gtp_sc, gtmin_tc, gm_sc, gm_tc):
    return pl.pallas_call(
        _combine_body,
        out_shape=jax.ShapeDtypeStruct((1, 1), jnp.float32),
        in_specs=[pl.BlockSpec(memory_space=pltpu.VMEM)] * 4,
        out_specs=pl.BlockSpec(memory_space=pltpu.SMEM),
    )(gtp_sc, gtmin_tc, gm_sc, gm_tc)


def kernel(gt_points_l, gen_points_l):
    gt = jnp.squeeze(gt_points_l)
    gen = jnp.squeeze(gen_points_l)

    gen_t = gen.T
    gt2_pad = jnp.pad((2.0 * gt).astype(jnp.bfloat16), ((0, 0), (0, 125)))
    genT_pad = jnp.pad(gen_t[:, :C_TC].astype(jnp.bfloat16), ((0, 125), (0, 0)))
    gt_pad8 = jnp.pad(gt, ((0, 0), (0, 5)))
    b2_row = jnp.sum(gen_t[:, :C_TC] * gen_t[:, :C_TC], axis=0)[None, :]

    gtp_sc, gm_sc = _sc_chamfer(gt.T, gen_t)
    gtmin_tc, gm_tc = _tc_chamfer(gt2_pad, genT_pad, gt_pad8, b2_row)

    out = _combine(gtp_sc.reshape(64, 128),
                   gtmin_tc.reshape(64, 128),
                   gm_sc.reshape(NW, SC_GEN // 128, 128),
                   gm_tc.reshape(C_TC // 128, 128))
    return out[0, 0]


# speedup vs baseline: 1.4249x; 1.4249x over previous
import functools

import jax
import jax.numpy as jnp
from jax import lax
from jax.experimental import pallas as pl
from jax.experimental.pallas import tpu as pltpu
from jax.experimental.pallas import tpu_sc as plsc

N = 8192
L = 16
NC = 2
NS = 16
NW = NC * NS

C_TC = 6656
SC_GEN = N - C_TC
NCH = SC_GEN // L
GT_PER_W = N // NW
NBLK_W = GT_PER_W // L
GSUB = 2
CGRP = 8

GT_TILE = 4096
GEN_TILE = 512

_F32_INF = 3.0e38

_GATHER_DNUMS = lax.GatherDimensionNumbers(
    offset_dims=(), collapsed_slice_dims=(0,), start_index_map=(0,))


def _permute16(x, idx):
    return lax.gather(x, idx[:, None], _GATHER_DNUMS, (1,),
                      indices_are_sorted=False, unique_indices=False,
                      mode=lax.GatherScatterMode.PROMISE_IN_BOUNDS)


def _all_lane_min(x, lane):
    for k in (8, 4, 2, 1):
        x = jnp.minimum(x, _permute16(x, lane ^ k))
    return x


def _rb_vec(x):
    u = lax.bitcast_convert_type(x, jnp.uint32)
    u = (u + jnp.uint32(0x7FFF) + ((u >> 16) & jnp.uint32(1))) \
        & jnp.uint32(0xFFFF0000)
    return lax.bitcast_convert_type(u, jnp.float32)


def _sc_body(gt_t, gen_t,
             gtp_out, genmin_out,
             rx_v, ry_v, rz_v,
             gs_s, a2_v, bx_v, by_v, bz_v, b2_v,
             gtp_v, gm_v):
    wid = lax.axis_index("s") * NC + lax.axis_index("c")
    base_t = wid * GT_PER_W

    pltpu.sync_copy(gen_t.at[pl.ds(0, 1), pl.ds(C_TC, SC_GEN)], rx_v)
    pltpu.sync_copy(gen_t.at[pl.ds(1, 1), pl.ds(C_TC, SC_GEN)], ry_v)
    pltpu.sync_copy(gen_t.at[pl.ds(2, 1), pl.ds(C_TC, SC_GEN)], rz_v)
    inf_vec = jnp.full((L,), _F32_INF, jnp.float32)

    def gen_stage(c, carry):
        sl = pl.ds(c * L, L)
        x, y, z = rx_v[0, sl], ry_v[0, sl], rz_v[0, sl]
        b2_v[sl] = x * x + y * y + z * z
        bx_v[sl] = _rb_vec(x)
        by_v[sl] = _rb_vec(y)
        bz_v[sl] = _rb_vec(z)
        gm_v[sl] = inf_vec
        return carry

    lax.fori_loop(0, NCH, gen_stage, 0)

    pltpu.sync_copy(gt_t.at[pl.ds(0, 1), pl.ds(base_t, GT_PER_W)],
                    rx_v.at[pl.ds(0, 1), pl.ds(0, GT_PER_W)])
    pltpu.sync_copy(gt_t.at[pl.ds(1, 1), pl.ds(base_t, GT_PER_W)],
                    ry_v.at[pl.ds(0, 1), pl.ds(0, GT_PER_W)])
    pltpu.sync_copy(gt_t.at[pl.ds(2, 1), pl.ds(base_t, GT_PER_W)],
                    rz_v.at[pl.ds(0, 1), pl.ds(0, GT_PER_W)])

    def gt_stage(c, carry):
        sl = pl.ds(c * L, L)
        x, y, z = rx_v[0, sl], ry_v[0, sl], rz_v[0, sl]
        a2 = x * x + y * y + z * z
        a2_v[sl] = a2
        gx = 2.0 * _rb_vec(x)
        gy = 2.0 * _rb_vec(y)
        gz = 2.0 * _rb_vec(z)
        base = c * L
        for g in range(L):
            gs_s[4 * (base + g) + 0] = gx[g]
            gs_s[4 * (base + g) + 1] = gy[g]
            gs_s[4 * (base + g) + 2] = gz[g]
            gs_s[4 * (base + g) + 3] = a2[g]
        return carry

    lax.fori_loop(0, NBLK_W, gt_stage, 0)

    lane = lax.iota(jnp.int32, L)

    def blk_body(blk, carry):
        base = blk * L
        a2v = a2_v[pl.ds(base, L)]
        r = inf_vec
        for s0 in range(0, L, GSUB):
            gs = range(s0, s0 + GSUB)
            ax = [gs_s[4 * (base + g) + 0] for g in gs]
            ay = [gs_s[4 * (base + g) + 1] for g in gs]
            az = [gs_s[4 * (base + g) + 2] for g in gs]
            a2 = [gs_s[4 * (base + g) + 3] for g in gs]

            def cg_body(cg, m):
                cbase = cg * (CGRP * L)
                for cc in range(CGRP):
                    off = cbase + cc * L
                    bx = bx_v[pl.ds(off, L)]
                    by = by_v[pl.ds(off, L)]
                    bz = bz_v[pl.ds(off, L)]
                    b2 = b2_v[pl.ds(off, L)]
                    gm = gm_v[pl.ds(off, L)]
                    for k in range(GSUB):
                        q = b2 - ax[k] * bx - ay[k] * by - az[k] * bz
                        gm = jnp.minimum(gm, q + a2[k])
                        m = tuple(
                            jnp.minimum(m[t], q) if t == k else m[t]
                            for t in range(GSUB))
                    gm_v[pl.ds(off, L)] = gm
                return m

            m = lax.fori_loop(0, NCH // CGRP, cg_body,
                              tuple(inf_vec for _ in range(GSUB)))
            for k in range(GSUB):
                r = jnp.where(lane == s0 + k, _all_lane_min(m[k], lane), r)
        gtp_v[pl.ds(base, L)] = r + a2v
        return carry

    lax.fori_loop(0, NBLK_W, blk_body, 0)

    pltpu.sync_copy(gtp_v, gtp_out.at[pl.ds(base_t, GT_PER_W)])
    pltpu.sync_copy(gm_v, genmin_out.at[wid])


def _sc_chamfer(gt_t, gen_t):
    mesh = plsc.VectorSubcoreMesh(core_axis_name="c", subcore_axis_name="s")
    f = pl.kernel(
        _sc_body,
        mesh=mesh,
        out_type=(
            jax.ShapeDtypeStruct((N,), jnp.float32),
            jax.ShapeDtypeStruct((NW, SC_GEN), jnp.float32),
        ),
        scratch_types=[
            pltpu.VMEM((1, SC_GEN), jnp.float32),
            pltpu.VMEM((1, SC_GEN), jnp.float32),
            pltpu.VMEM((1, SC_GEN), jnp.float32),
            pltpu.SMEM((4 * GT_PER_W,), jnp.float32),
            pltpu.VMEM((GT_PER_W,), jnp.float32),
            pltpu.VMEM((SC_GEN,), jnp.float32),
            pltpu.VMEM((SC_GEN,), jnp.float32),
            pltpu.VMEM((SC_GEN,), jnp.float32),
            pltpu.VMEM((SC_GEN,), jnp.float32),
            pltpu.VMEM((GT_PER_W,), jnp.float32),
            pltpu.VMEM((SC_GEN,), jnp.float32),
        ],
    )
    return f(gt_t, gen_t)


def _tc_body(gt2_ref, genT_ref, a2_ref, b2_ref, gtmin_ref, genmin_ref,
             acc_gt, acc_gen):
    j = pl.program_id(0)
    i = pl.program_id(1)
    ab2 = jnp.dot(gt2_ref[...], genT_ref[...],
                  preferred_element_type=jnp.float32)
    a2c = a2_ref[...]
    b2r = b2_ref[...]
    q = b2r - ab2
    p = a2c - ab2
    mq = jnp.min(q, axis=1, keepdims=True) + a2c
    mp = jnp.min(p, axis=0, keepdims=True) + b2r

    gt_sl = pl.ds(j * GT_TILE, GT_TILE)
    gen_sl = pl.ds(i * GEN_TILE, GEN_TILE)

    @pl.when(i == 0)
    def _():
        acc_gt[gt_sl, :] = mq

    @pl.when(i > 0)
    def _():
        acc_gt[gt_sl, :] = jnp.minimum(acc_gt[gt_sl, :], mq)

    @pl.when(j == 0)
    def _():
        acc_gen[:, gen_sl] = mp

    @pl.when(j > 0)
    def _():
        acc_gen[:, gen_sl] = jnp.minimum(acc_gen[:, gen_sl], mp)

    @pl.when((j == N // GT_TILE - 1) & (i == C_TC // GEN_TILE - 1))
    def _():
        gtmin_ref[...] = jnp.reshape(acc_gt[...], (64, 128))
        genmin_ref[...] = jnp.reshape(acc_gen[...], (C_TC // 128, 128))


def _tc_chamfer(gt2_pad, genT_pad, a2_col, b2_row):
    return pl.pallas_call(
        _tc_body,
        grid=(N // GT_TILE, C_TC // GEN_TILE),
        in_specs=[
            pl.BlockSpec((GT_TILE, 128), lambda j, i: (j, 0)),
            pl.BlockSpec((128, GEN_TILE), lambda j, i: (0, i)),
            pl.BlockSpec((GT_TILE, 1), lambda j, i: (j, 0)),
            pl.BlockSpec((1, GEN_TILE), lambda j, i: (0, i)),
        ],
        out_specs=[
            pl.BlockSpec((64, 128), lambda j, i: (0, 0)),
            pl.BlockSpec((C_TC // 128, 128), lambda j, i: (0, 0)),
        ],
        out_shape=[
            jax.ShapeDtypeStruct((64, 128), jnp.float32),
            jax.ShapeDtypeStruct((C_TC // 128, 128), jnp.float32),
        ],
        scratch_shapes=[
            pltpu.VMEM((N, 1), jnp.float32),
            pltpu.VMEM((1, C_TC), jnp.float32),
        ],
    )(gt2_pad, genT_pad, a2_col, b2_row)


def _combine_body(gtp_sc_ref, gtmin_tc_ref, gm_sc_ref, gm_tc_ref, out_ref):
    gt_full = jnp.minimum(gtp_sc_ref[...], gtmin_tc_ref[...])
    gen_sc = jnp.min(gm_sc_ref[...], axis=0)
    s = (jnp.sum(jnp.maximum(gt_full, 0.0))
         + jnp.sum(jnp.maximum(gen_sc, 0.0))
         + jnp.sum(jnp.maximum(gm_tc_ref[...], 0.0)))
    out_ref[0, 0] = s * (1.0 / N)


def _combine(gtp_sc, gtmin_tc, gm_sc, gm_tc):
    return pl.pallas_call(
        _combine_body,
        out_shape=jax.ShapeDtypeStruct((1, 1), jnp.float32),
        in_specs=[pl.BlockSpec(memory_space=pltpu.VMEM)] * 4,
        out_specs=pl.BlockSpec(memory_space=pltpu.SMEM),
    )(gtp_sc, gtmin_tc, gm_sc, gm_tc)


def kernel(gt_points_l, gen_points_l):
    gt = jnp.squeeze(gt_points_l)
    gen = jnp.squeeze(gen_points_l)

    gen_t = gen.T
    gt2_pad = jnp.pad((2.0 * gt).astype(jnp.bfloat16), ((0, 0), (0, 125)))
    genT_pad = jnp.pad(gen_t[:, :C_TC].astype(jnp.bfloat16), ((0, 125), (0, 0)))
    a2_col = jnp.sum(gt * gt, axis=1, keepdims=True)
    b2_row = jnp.sum(gen_t[:, :C_TC] * gen_t[:, :C_TC], axis=0)[None, :]

    gtp_sc, gm_sc = _sc_chamfer(gt.T, gen_t)
    gtmin_tc, gm_tc = _tc_chamfer(gt2_pad, genT_pad, a2_col, b2_row)

    out = _combine(gtp_sc.reshape(64, 128),
                   gtmin_tc.reshape(64, 128),
                   gm_sc.reshape(NW, SC_GEN // 128, 128),
                   gm_tc.reshape(C_TC // 128, 128))
    return out[0, 0]
